# baseline (speedup 1.0000x reference)
import functools

import jax
import jax.numpy as jnp
from jax import lax
from jax.experimental import pallas as pl
from jax.experimental.pallas import tpu as pltpu

D = 768
E = 8
NBUF = 8
BLK = 512


def _tc_body(x_hbm, wt_ref, o_ref, bufs, sems):
    T = x_hbm.shape[0]
    nblk = T // BLK
    wt = wt_ref[...]

    def start(i, p):
        pltpu.make_async_copy(
            x_hbm.at[pl.ds(i * BLK, BLK)], bufs.at[p], sems.at[p]).start()

    for b in range(NBUF):
        start(b, b)

    def step(i, _):
        p = lax.rem(i, NBUF)
        pltpu.make_async_copy(
            x_hbm.at[pl.ds(i * BLK, BLK)], bufs.at[p], sems.at[p]).wait()
        o_ref[pl.ds(i * BLK, BLK)] = jnp.dot(
            bufs[p], wt, preferred_element_type=jnp.float32)

        @pl.when(i + NBUF < nblk)
        def _():
            start(i + NBUF, p)

        return 0

    lax.fori_loop(0, nblk, step, 0)


def kernel(x, W):
    T = x.shape[0]
    Wt = W.T
    out = pl.pallas_call(
        _tc_body,
        in_specs=[
            pl.BlockSpec(memory_space=pl.ANY),
            pl.BlockSpec(memory_space=pltpu.VMEM),
        ],
        out_specs=pl.BlockSpec(memory_space=pltpu.VMEM),
        out_shape=jax.ShapeDtypeStruct((T, E), jnp.float32),
        scratch_shapes=[
            pltpu.VMEM((NBUF, BLK, D), jnp.float32),
            pltpu.SemaphoreType.DMA((NBUF,)),
        ],
    )(x, Wt)
    return out
